# Initial kernel scaffold; baseline (speedup 1.0000x reference)
import jax
import jax.numpy as jnp
from jax.experimental import pallas as pl


def kernel(x, W0, b0, We, be, Wl, bl):
    raise NotImplementedError("write your pallas kernel here")


# speedup vs baseline: 45.7202x; 45.7202x over previous
import functools

import jax
import jax.numpy as jnp
from jax.experimental import pallas as pl
from jax.experimental.pallas import tpu as pltpu

_DEPTH = 6


def _encoder_body(xc_ref, xt_ref, W0_ref, b0_ref, We_ref, be_ref,
                  Wl_ref, bl_ref, out_ref, *, n, em, depth):
    f32 = jnp.float32
    xg = xc_ref[0]
    x0c = xc_ref[0, :, 0:1]
    x1c = xc_ref[0, :, 1:2]
    x0r = xt_ref[0, 0:1, :]
    x1r = xt_ref[0, 1:2, :]
    d2 = (x0c - x0r) ** 2 + (x1c - x1r) ** 2
    D = jnp.sqrt(d2 + 1e-12)

    ii = jax.lax.broadcasted_iota(jnp.int32, (n, n), 0)
    jj = jax.lax.broadcasted_iota(jnp.int32, (n, n), 1)
    offdiag = (ii != jj).astype(f32)

    We = We_ref[...]
    be = be_ref[...]
    pre = D[:, :, None] * We[None] + be[None]
    E = jax.nn.silu(pre) * offdiag[:, :, None]

    h = jax.nn.silu(
        jnp.dot(xg, W0_ref[...], preferred_element_type=f32) + b0_ref[...])
    for l in range(depth):
        agg = jnp.sum(E * h[:, None, :], axis=0)
        h = jax.nn.silu(
            jnp.dot(agg, Wl_ref[l], preferred_element_type=f32)
            + bl_ref[l]) + h
    out_ref[0] = jnp.sum(h, axis=0, keepdims=True) * (1.0 / n)


@jax.jit
def kernel(x, W0, b0, We, be, Wl, bl):
    seq_len, batch, n, nf = x.shape
    G = seq_len * batch
    em = W0.shape[1]
    xc = x.reshape(G, n, nf)
    xt = xc.transpose(0, 2, 1)
    b0r = b0.reshape(1, em)
    ber = be.reshape(1, em)

    body = functools.partial(_encoder_body, n=n, em=em, depth=_DEPTH)
    out = pl.pallas_call(
        body,
        grid=(G,),
        in_specs=[
            pl.BlockSpec((1, n, nf), lambda g: (g, 0, 0)),
            pl.BlockSpec((1, nf, n), lambda g: (g, 0, 0)),
            pl.BlockSpec((nf, em), lambda g: (0, 0)),
            pl.BlockSpec((1, em), lambda g: (0, 0)),
            pl.BlockSpec((1, em), lambda g: (0, 0)),
            pl.BlockSpec((1, em), lambda g: (0, 0)),
            pl.BlockSpec((_DEPTH, em, em), lambda g: (0, 0, 0)),
            pl.BlockSpec((_DEPTH, em), lambda g: (0, 0)),
        ],
        out_specs=pl.BlockSpec((1, 1, em), lambda g: (g, 0, 0)),
        out_shape=jax.ShapeDtypeStruct((G, 1, em), jnp.float32),
    )(xc, xt, W0, b0r, We, ber, Wl, bl)
    return out.reshape(seq_len, batch, em)


# speedup vs baseline: 74.2631x; 1.6243x over previous
import functools

import jax
import jax.numpy as jnp
from jax.experimental import pallas as pl
from jax.experimental.pallas import tpu as pltpu

_DEPTH = 6


def _encoder_body(xc_ref, xt_ref, W0_ref, b0_ref, We2_ref, be2_ref,
                  Wl2_ref, bl2_ref, out_ref, *, n, em, depth):
    f32 = jnp.float32
    We2 = We2_ref[...]
    be2 = be2_ref[...]

    def dmat(g):
        x0c = xc_ref[g, :, 0:1]
        x1c = xc_ref[g, :, 1:2]
        x0r = xt_ref[g, 0:1, :]
        x1r = xt_ref[g, 1:2, :]
        d2 = (x0c - x0r) ** 2 + (x1c - x1r) ** 2
        return jnp.sqrt(d2 + 1e-12)

    DA = dmat(0)
    DB = dmat(1)
    We1 = We2[:, :em]
    be1 = be2[:, :em]
    pre = jnp.concatenate(
        [DA[:, :, None] * We1[None] + be1[None],
         DB[:, :, None] * We1[None] + be1[None]], axis=2)
    E = jax.nn.silu(pre)
    sdiag = jax.nn.silu(1e-6 * We2 + be2)

    hA = jax.nn.silu(
        jnp.dot(xc_ref[0], W0_ref[...], preferred_element_type=f32)
        + b0_ref[...])
    hB = jax.nn.silu(
        jnp.dot(xc_ref[1], W0_ref[...], preferred_element_type=f32)
        + b0_ref[...])
    h = jnp.concatenate([hA, hB], axis=1)
    for l in range(depth):
        agg = jnp.sum(E * h[:, None, :], axis=0) - sdiag * h
        h = jax.nn.silu(
            jnp.dot(agg, Wl2_ref[l], preferred_element_type=f32)
            + bl2_ref[l]) + h
    out_ref[0] = jnp.sum(h, axis=0, keepdims=True) * (1.0 / n)


@jax.jit
def kernel(x, W0, b0, We, be, Wl, bl):
    seq_len, batch, n, nf = x.shape
    G = seq_len * batch
    em = W0.shape[1]
    depth = Wl.shape[0]
    xc = x.reshape(G, n, nf)
    xt = xc.transpose(0, 2, 1)
    b0r = b0.reshape(1, em)
    We2 = jnp.concatenate([We.reshape(1, em)] * 2, axis=1)
    be2 = jnp.concatenate([be.reshape(1, em)] * 2, axis=1)
    Wl2 = jnp.zeros((depth, 2 * em, 2 * em), jnp.float32)
    Wl2 = Wl2.at[:, :em, :em].set(Wl).at[:, em:, em:].set(Wl)
    bl2 = jnp.concatenate([bl, bl], axis=1)

    body = functools.partial(_encoder_body, n=n, em=em, depth=depth)
    out = pl.pallas_call(
        body,
        grid=(G // 2,),
        in_specs=[
            pl.BlockSpec((2, n, nf), lambda g: (g, 0, 0)),
            pl.BlockSpec((2, nf, n), lambda g: (g, 0, 0)),
            pl.BlockSpec((nf, em), lambda g: (0, 0)),
            pl.BlockSpec((1, em), lambda g: (0, 0)),
            pl.BlockSpec((1, 2 * em), lambda g: (0, 0)),
            pl.BlockSpec((1, 2 * em), lambda g: (0, 0)),
            pl.BlockSpec((depth, 2 * em, 2 * em), lambda g: (0, 0, 0)),
            pl.BlockSpec((depth, 2 * em), lambda g: (0, 0)),
        ],
        out_specs=pl.BlockSpec((1, 1, 2 * em), lambda g: (g, 0, 0)),
        out_shape=jax.ShapeDtypeStruct((G // 2, 1, 2 * em), jnp.float32),
    )(xc, xt, W0, b0r, We2, be2, Wl2, bl2)
    return out.reshape(seq_len, batch, em)


# speedup vs baseline: 99.3322x; 1.3376x over previous
import functools

import jax
import jax.numpy as jnp
from jax.experimental import pallas as pl
from jax.experimental.pallas import tpu as pltpu

_DEPTH = 6


def _silu(x):
    return (0.5 * x) * (1.0 + jnp.tanh(0.5 * x))


def _encoder_body(xc_ref, xt_ref, W0_ref, b0_ref, We2_ref, be2_ref,
                  Wek_ref, Wl2_ref, bl2_ref, out_ref, *, n, em, depth):
    f32 = jnp.float32
    We2 = We2_ref[...]
    be2 = be2_ref[...]

    def dmat(g):
        x0c = xc_ref[g, :, 0:1]
        x1c = xc_ref[g, :, 1:2]
        x0r = xt_ref[g, 0:1, :]
        x1r = xt_ref[g, 1:2, :]
        d2 = (x0c - x0r) ** 2 + (x1c - x1r) ** 2
        return jnp.sqrt(d2 + 1e-12)

    DA = dmat(0)
    DB = dmat(1)
    Wek = Wek_ref[...]
    ones_row = jnp.full((1, n), 1.0, f32)
    planes = []
    for i in range(n):
        lhs = jnp.concatenate(
            [DA[i:i + 1, :], DB[i:i + 1, :], ones_row], axis=0)
        planes.append(jax.lax.dot_general(
            lhs, Wek, (((0,), (0,)), ((), ())),
            preferred_element_type=f32)[None])
    pre = jnp.concatenate(planes, axis=0)
    E = _silu(pre)
    sdiag = _silu(1e-6 * We2 + be2)

    hA = _silu(
        jnp.dot(xc_ref[0], W0_ref[...], preferred_element_type=f32)
        + b0_ref[...])
    hB = _silu(
        jnp.dot(xc_ref[1], W0_ref[...], preferred_element_type=f32)
        + b0_ref[...])
    h = jnp.concatenate([hA, hB], axis=1)
    for l in range(depth):
        agg = jnp.sum(E * h[:, None, :], axis=0) - sdiag * h
        h = _silu(
            jnp.dot(agg, Wl2_ref[l], preferred_element_type=f32)
            + bl2_ref[l]) + h
    out_ref[0] = jnp.sum(h, axis=0, keepdims=True) * (1.0 / n)


@jax.jit
def kernel(x, W0, b0, We, be, Wl, bl):
    seq_len, batch, n, nf = x.shape
    G = seq_len * batch
    em = W0.shape[1]
    depth = Wl.shape[0]
    xc = x.reshape(G, n, nf)
    xt = xc.transpose(0, 2, 1)
    b0r = b0.reshape(1, em)
    We2 = jnp.concatenate([We.reshape(1, em)] * 2, axis=1)
    be2 = jnp.concatenate([be.reshape(1, em)] * 2, axis=1)
    Wl2 = jnp.zeros((depth, 2 * em, 2 * em), jnp.float32)
    Wl2 = Wl2.at[:, :em, :em].set(Wl).at[:, em:, em:].set(Wl)
    bl2 = jnp.concatenate([bl, bl], axis=1)
    zer = jnp.zeros((1, em), jnp.float32)
    Wek = jnp.concatenate([
        jnp.concatenate([We.reshape(1, em), zer], axis=1),
        jnp.concatenate([zer, We.reshape(1, em)], axis=1),
        be2], axis=0)

    body = functools.partial(_encoder_body, n=n, em=em, depth=depth)
    out = pl.pallas_call(
        body,
        grid=(G // 2,),
        in_specs=[
            pl.BlockSpec((2, n, nf), lambda g: (g, 0, 0)),
            pl.BlockSpec((2, nf, n), lambda g: (g, 0, 0)),
            pl.BlockSpec((nf, em), lambda g: (0, 0)),
            pl.BlockSpec((1, em), lambda g: (0, 0)),
            pl.BlockSpec((1, 2 * em), lambda g: (0, 0)),
            pl.BlockSpec((1, 2 * em), lambda g: (0, 0)),
            pl.BlockSpec((3, 2 * em), lambda g: (0, 0)),
            pl.BlockSpec((depth, 2 * em, 2 * em), lambda g: (0, 0, 0)),
            pl.BlockSpec((depth, 2 * em), lambda g: (0, 0)),
        ],
        out_specs=pl.BlockSpec((1, 1, 2 * em), lambda g: (g, 0, 0)),
        out_shape=jax.ShapeDtypeStruct((G // 2, 1, 2 * em), jnp.float32),
    )(xc, xt, W0, b0r, We2, be2, Wek, Wl2, bl2)
    return out.reshape(seq_len, batch, em)
